# Initial kernel scaffold; baseline (speedup 1.0000x reference)
import jax
import jax.numpy as jnp
from jax import lax
from jax.experimental import pallas as pl
from jax.experimental.pallas import tpu as pltpu
from jax.experimental.pallas import tpu_sc as plsc

MARGIN = 0.2
EPS = 1e-6
N = 4096
D = 128
BR = 512
BC = 1024
LANES = 16
NC = 2
NS = 16
NW = NC * NS
BPW = N // NW


def _mine_body(frow, fcol, idrow, idcol, pos_out, neg_out,
               pmax, pidx, nmin, nidx):
    c = pl.program_id(1)

    @pl.when(c == 0)
    def _init():
        pmax[:, :] = jnp.full((BR, 1), -1.0, jnp.float32)
        pidx[:, :] = jnp.zeros((BR, 1), jnp.int32)
        nmin[:, :] = jnp.full((BR, 1), jnp.inf, jnp.float32)
        nidx[:, :] = jnp.zeros((BR, 1), jnp.int32)

    a = frow[:, :]
    b = fcol[:, :]
    sqa = jnp.sum(a * a, axis=1, keepdims=True)
    ones = jnp.ones((1, D), jnp.float32)
    sqb = lax.dot_general(ones, b * b, (((1,), (1,)), ((), ())),
                          preferred_element_type=jnp.float32)
    dot = lax.dot_general(a, b, (((1,), (1,)), ((), ())),
                          preferred_element_type=jnp.float32,
                          precision=lax.Precision.HIGHEST)
    d2 = jnp.maximum(sqa + sqb - 2.0 * dot, 0.0)
    same = idrow[:, :] == idcol[:, :]
    jglob = c * BC + lax.broadcasted_iota(jnp.int32, (BR, BC), 1)
    big = jnp.int32(2**30)

    posv = jnp.where(same, d2, 0.0)
    pm = jnp.max(posv, axis=1, keepdims=True)
    pj = jnp.min(jnp.where(posv == pm, jglob, big), axis=1, keepdims=True)
    pu = pm > pmax[:, :]
    pidx[:, :] = jnp.where(pu, pj, pidx[:, :])
    pmax[:, :] = jnp.where(pu, pm, pmax[:, :])

    negv = jnp.where(same, jnp.inf, d2)
    nm = jnp.min(negv, axis=1, keepdims=True)
    nj = jnp.min(jnp.where(negv == nm, jglob, big), axis=1, keepdims=True)
    nu = nm < nmin[:, :]
    nidx[:, :] = jnp.where(nu, nj, nidx[:, :])
    nmin[:, :] = jnp.where(nu, nm, nmin[:, :])

    pos_out[:, :] = pidx[:, :]
    neg_out[:, :] = nidx[:, :]


_mine = pl.pallas_call(
    _mine_body,
    grid=(N // BR, N // BC),
    in_specs=[
        pl.BlockSpec((BR, D), lambda r, c: (r, 0)),
        pl.BlockSpec((BC, D), lambda r, c: (c, 0)),
        pl.BlockSpec((BR, 1), lambda r, c: (r, 0)),
        pl.BlockSpec((1, BC), lambda r, c: (0, c)),
    ],
    out_specs=[
        pl.BlockSpec((BR, 1), lambda r, c: (r, 0)),
        pl.BlockSpec((BR, 1), lambda r, c: (r, 0)),
    ],
    out_shape=[
        jax.ShapeDtypeStruct((N, 1), jnp.int32),
        jax.ShapeDtypeStruct((N, 1), jnp.int32),
    ],
    scratch_shapes=[
        pltpu.VMEM((BR, 1), jnp.float32),
        pltpu.VMEM((BR, 1), jnp.int32),
        pltpu.VMEM((BR, 1), jnp.float32),
        pltpu.VMEM((BR, 1), jnp.int32),
    ],
)


def _sc_body(feat_hbm, pidx_hbm, nidx_hbm, outp_hbm, outn_hbm,
             pidx_v, nidx_v, self_v, pos_v, neg_v, accp_v, accn_v,
             sem_p, sem_n):
    wid = lax.axis_index("s") * NC + lax.axis_index("c")
    base = wid * BPW
    pltpu.sync_copy(pidx_hbm.at[pl.ds(base, BPW)], pidx_v)
    pltpu.sync_copy(nidx_hbm.at[pl.ds(base, BPW)], nidx_v)
    cp = pltpu.async_copy(feat_hbm.at[pidx_v], pos_v, sem_p)
    cn = pltpu.async_copy(feat_hbm.at[nidx_v], neg_v, sem_n)
    pltpu.sync_copy(feat_hbm.at[pl.ds(base, BPW)], self_v)
    cp.wait()
    cn.wait()

    def row(i, carry):
        ap = jnp.zeros((LANES,), jnp.float32)
        an = jnp.zeros((LANES,), jnp.float32)
        for ch in range(D // LANES):
            s = self_v[i, pl.ds(ch * LANES, LANES)]
            p = pos_v[i, pl.ds(ch * LANES, LANES)]
            m = neg_v[i, pl.ds(ch * LANES, LANES)]
            dp = s - p + EPS
            dn = s - m + EPS
            ap = ap + dp * dp
            an = an + dn * dn
        accp_v[i, :] = ap
        accn_v[i, :] = an
        return carry

    lax.fori_loop(0, BPW, row, 0)
    pltpu.sync_copy(accp_v, outp_hbm.at[pl.ds(base, BPW)])
    pltpu.sync_copy(accn_v, outn_hbm.at[pl.ds(base, BPW)])


_sc_gather = pl.kernel(
    _sc_body,
    out_type=[
        jax.ShapeDtypeStruct((N, LANES), jnp.float32),
        jax.ShapeDtypeStruct((N, LANES), jnp.float32),
    ],
    mesh=plsc.VectorSubcoreMesh(core_axis_name="c", subcore_axis_name="s"),
    scratch_types=[
        pltpu.VMEM((BPW,), jnp.int32),
        pltpu.VMEM((BPW,), jnp.int32),
        pltpu.VMEM((BPW, D), jnp.float32),
        pltpu.VMEM((BPW, D), jnp.float32),
        pltpu.VMEM((BPW, D), jnp.float32),
        pltpu.VMEM((BPW, LANES), jnp.float32),
        pltpu.VMEM((BPW, LANES), jnp.float32),
        pltpu.SemaphoreType.DMA,
        pltpu.SemaphoreType.DMA,
    ],
)


def _finish_body(d2p_ref, d2n_ref, out_ref):
    dp = jnp.sqrt(jnp.sum(d2p_ref[:, :], axis=1, keepdims=True))
    dn = jnp.sqrt(jnp.sum(d2n_ref[:, :], axis=1, keepdims=True))
    h = jnp.maximum(MARGIN + dp - dn, 0.0)
    out_ref[0, 0] = jnp.sum(h) / N


_finish = pl.pallas_call(
    _finish_body,
    out_shape=jax.ShapeDtypeStruct((1, 1), jnp.float32),
)


def kernel(feature, identity):
    ident = identity.astype(jnp.int32)
    pos2, neg2 = _mine(feature, feature, ident.reshape(N, 1),
                       ident.reshape(1, N))
    d2p, d2n = _sc_gather(feature, pos2.reshape(N), neg2.reshape(N))
    return _finish(d2p, d2n)[0, 0]


# speedup vs baseline: 1.0837x; 1.0837x over previous
import functools

import jax
import jax.numpy as jnp
from jax import lax
from jax.experimental import pallas as pl
from jax.experimental.pallas import tpu as pltpu
from jax.experimental.pallas import tpu_sc as plsc

MARGIN = 0.2
EPS = 1e-6
N = 4096
D = 128
BR = 512
BC = 1024
LANES = 16
NC = 2
NS = 16
NW = NC * NS
BPW = N // NW


def _mine_body(frow, fcol, idrow, idcol, pos_out, neg_out,
               pmax, pidx, nmin, nidx):
    c = pl.program_id(1)

    @pl.when(c == 0)
    def _init():
        pmax[:, :] = jnp.full((BR, 1), -1.0, jnp.float32)
        pidx[:, :] = jnp.zeros((BR, 1), jnp.int32)
        nmin[:, :] = jnp.full((BR, 1), jnp.inf, jnp.float32)
        nidx[:, :] = jnp.zeros((BR, 1), jnp.int32)

    a = frow[:, :]
    b = fcol[:, :]
    sqa = jnp.sum(a * a, axis=1, keepdims=True)
    ones = jnp.ones((1, D), jnp.float32)
    sqb = lax.dot_general(ones, b * b, (((1,), (1,)), ((), ())),
                          preferred_element_type=jnp.float32)
    dot = lax.dot_general(a, b, (((1,), (1,)), ((), ())),
                          preferred_element_type=jnp.float32,
                          precision=lax.Precision.HIGHEST)
    d2 = jnp.maximum(sqa + sqb - 2.0 * dot, 0.0)
    same = idrow[:, :] == idcol[:, :]
    jglob = c * BC + lax.broadcasted_iota(jnp.int32, (BR, BC), 1)
    big = jnp.int32(2**30)

    posv = jnp.where(same, d2, 0.0)
    pm = jnp.max(posv, axis=1, keepdims=True)
    pj = jnp.min(jnp.where(posv == pm, jglob, big), axis=1, keepdims=True)
    pu = pm > pmax[:, :]
    pidx[:, :] = jnp.where(pu, pj, pidx[:, :])
    pmax[:, :] = jnp.where(pu, pm, pmax[:, :])

    negv = jnp.where(same, jnp.inf, d2)
    nm = jnp.min(negv, axis=1, keepdims=True)
    nj = jnp.min(jnp.where(negv == nm, jglob, big), axis=1, keepdims=True)
    nu = nm < nmin[:, :]
    nidx[:, :] = jnp.where(nu, nj, nidx[:, :])
    nmin[:, :] = jnp.where(nu, nm, nmin[:, :])

    pos_out[:, :] = pidx[:, :]
    neg_out[:, :] = nidx[:, :]


_mine = pl.pallas_call(
    _mine_body,
    grid=(N // BR, N // BC),
    in_specs=[
        pl.BlockSpec((BR, D), lambda r, c: (r, 0)),
        pl.BlockSpec((BC, D), lambda r, c: (c, 0)),
        pl.BlockSpec((BR, 1), lambda r, c: (r, 0)),
        pl.BlockSpec((1, BC), lambda r, c: (0, c)),
    ],
    out_specs=[
        pl.BlockSpec((BR, 1), lambda r, c: (r, 0)),
        pl.BlockSpec((BR, 1), lambda r, c: (r, 0)),
    ],
    out_shape=[
        jax.ShapeDtypeStruct((N, 1), jnp.int32),
        jax.ShapeDtypeStruct((N, 1), jnp.int32),
    ],
    scratch_shapes=[
        pltpu.VMEM((BR, 1), jnp.float32),
        pltpu.VMEM((BR, 1), jnp.int32),
        pltpu.VMEM((BR, 1), jnp.float32),
        pltpu.VMEM((BR, 1), jnp.int32),
    ],
)


def _sc_body(feat_hbm, pidx_hbm, nidx_hbm, outp_hbm, outn_hbm,
             pidx_v, nidx_v, self_v, pos_v, neg_v, accp_v, accn_v,
             sem_p, sem_n):
    wid = lax.axis_index("s") * NC + lax.axis_index("c")
    base = wid * BPW
    pltpu.sync_copy(pidx_hbm.at[pl.ds(base, BPW)], pidx_v)
    pltpu.sync_copy(nidx_hbm.at[pl.ds(base, BPW)], nidx_v)
    cp = pltpu.async_copy(feat_hbm.at[pidx_v], pos_v, sem_p)
    cn = pltpu.async_copy(feat_hbm.at[nidx_v], neg_v, sem_n)
    pltpu.sync_copy(feat_hbm.at[pl.ds(base, BPW)], self_v)
    cp.wait()
    cn.wait()

    def row(i, carry):
        ap = jnp.zeros((LANES,), jnp.float32)
        an = jnp.zeros((LANES,), jnp.float32)
        for ch in range(D // LANES):
            s = self_v[i, pl.ds(ch * LANES, LANES)]
            p = pos_v[i, pl.ds(ch * LANES, LANES)]
            m = neg_v[i, pl.ds(ch * LANES, LANES)]
            dp = s - p + EPS
            dn = s - m + EPS
            ap = ap + dp * dp
            an = an + dn * dn
        accp_v[i, :] = ap
        accn_v[i, :] = an
        return carry

    lax.fori_loop(0, BPW, row, 0)
    pltpu.sync_copy(accp_v, outp_hbm.at[pl.ds(base, BPW)])
    pltpu.sync_copy(accn_v, outn_hbm.at[pl.ds(base, BPW)])


@functools.lru_cache(maxsize=1)
def _sc_gather():
  return pl.kernel(
    _sc_body,
    out_type=[
        jax.ShapeDtypeStruct((N, LANES), jnp.float32),
        jax.ShapeDtypeStruct((N, LANES), jnp.float32),
    ],
    mesh=plsc.VectorSubcoreMesh(core_axis_name="c", subcore_axis_name="s"),
    scratch_types=[
        pltpu.VMEM((BPW,), jnp.int32),
        pltpu.VMEM((BPW,), jnp.int32),
        pltpu.VMEM((BPW, D), jnp.float32),
        pltpu.VMEM((BPW, D), jnp.float32),
        pltpu.VMEM((BPW, D), jnp.float32),
        pltpu.VMEM((BPW, LANES), jnp.float32),
        pltpu.VMEM((BPW, LANES), jnp.float32),
        pltpu.SemaphoreType.DMA,
        pltpu.SemaphoreType.DMA,
    ],
  )


def _finish_body(d2p_ref, d2n_ref, out_ref):
    dp = jnp.sqrt(jnp.sum(d2p_ref[:, :], axis=1, keepdims=True))
    dn = jnp.sqrt(jnp.sum(d2n_ref[:, :], axis=1, keepdims=True))
    h = jnp.maximum(MARGIN + dp - dn, 0.0)
    out_ref[0, 0] = jnp.sum(h) / N


_finish = pl.pallas_call(
    _finish_body,
    out_specs=pl.BlockSpec(memory_space=pltpu.SMEM),
    out_shape=jax.ShapeDtypeStruct((1, 1), jnp.float32),
)


def kernel(feature, identity):
    ident = identity.astype(jnp.int32)
    pos2, neg2 = _mine(feature, feature, ident.reshape(N, 1),
                       ident.reshape(1, N))
    d2p, d2n = _sc_gather()(feature, pos2.reshape(N), neg2.reshape(N))
    return _finish(d2p, d2n)[0, 0]


# speedup vs baseline: 1.5341x; 1.4155x over previous
import functools

import jax
import jax.numpy as jnp
from jax import lax
from jax.experimental import pallas as pl
from jax.experimental.pallas import tpu as pltpu
from jax.experimental.pallas import tpu_sc as plsc

MARGIN = 0.2
EPS = 1e-6
N = 4096
D = 128
BR = 512
BC = 1024
LANES = 16
NC = 2
NS = 16
NW = NC * NS
BPW = N // NW


BIG = 3e38


def _mine_body(frow, fcol, idrow, idcol, pos_out, neg_out,
               pmax, pidx, nmin, nidx):
    c = pl.program_id(1)

    @pl.when(c == 0)
    def _init():
        pmax[:, :] = jnp.full((BR, 1), -BIG, jnp.float32)
        pidx[:, :] = jnp.zeros((BR, 1), jnp.int32)
        nmin[:, :] = jnp.full((BR, 1), BIG, jnp.float32)
        nidx[:, :] = jnp.zeros((BR, 1), jnp.int32)

    a = frow[:, :]
    b = fcol[:, :]
    ones = jnp.ones((1, D), jnp.float32)
    sqb = lax.dot_general(ones, b * b, (((1,), (1,)), ((), ())),
                          preferred_element_type=jnp.float32)
    dot = lax.dot_general(a.astype(jnp.bfloat16), b.astype(jnp.bfloat16),
                          (((1,), (1,)), ((), ())),
                          preferred_element_type=jnp.float32)
    u = sqb - 2.0 * dot
    same = idrow[:, :] == idcol[:, :]
    jglob = c * BC + lax.broadcasted_iota(jnp.int32, (BR, BC), 1)
    bigi = jnp.int32(2**30)

    posu = jnp.where(same, u, -BIG)
    um = jnp.max(posu, axis=1, keepdims=True)
    pj = jnp.min(jnp.where(posu == um, jglob, bigi), axis=1, keepdims=True)
    pu = um > pmax[:, :]
    pidx[:, :] = jnp.where(pu, pj, pidx[:, :])
    pmax[:, :] = jnp.where(pu, um, pmax[:, :])

    negu = jnp.where(same, BIG, u)
    nm = jnp.min(negu, axis=1, keepdims=True)
    nj = jnp.min(jnp.where(negu == nm, jglob, bigi), axis=1, keepdims=True)
    nu = nm < nmin[:, :]
    nidx[:, :] = jnp.where(nu, nj, nidx[:, :])
    nmin[:, :] = jnp.where(nu, nm, nmin[:, :])

    sqa = jnp.sum(a * a, axis=1, keepdims=True)
    pos_out[:, :] = jnp.where(pmax[:, :] + sqa > 0.0, pidx[:, :], 0)
    neg_out[:, :] = nidx[:, :]


_mine = pl.pallas_call(
    _mine_body,
    grid=(N // BR, N // BC),
    in_specs=[
        pl.BlockSpec((BR, D), lambda r, c: (r, 0)),
        pl.BlockSpec((BC, D), lambda r, c: (c, 0)),
        pl.BlockSpec((BR, 1), lambda r, c: (r, 0)),
        pl.BlockSpec((1, BC), lambda r, c: (0, c)),
    ],
    out_specs=[
        pl.BlockSpec((BR, 1), lambda r, c: (r, 0)),
        pl.BlockSpec((BR, 1), lambda r, c: (r, 0)),
    ],
    out_shape=[
        jax.ShapeDtypeStruct((N, 1), jnp.int32),
        jax.ShapeDtypeStruct((N, 1), jnp.int32),
    ],
    scratch_shapes=[
        pltpu.VMEM((BR, 1), jnp.float32),
        pltpu.VMEM((BR, 1), jnp.int32),
        pltpu.VMEM((BR, 1), jnp.float32),
        pltpu.VMEM((BR, 1), jnp.int32),
    ],
)


def _sc_body(feat_hbm, pidx_hbm, nidx_hbm, outp_hbm, outn_hbm,
             pidx_v, nidx_v, self_v, pos_v, neg_v, accp_v, accn_v,
             sem_p, sem_n):
    wid = lax.axis_index("s") * NC + lax.axis_index("c")
    base = wid * BPW
    pltpu.sync_copy(pidx_hbm.at[pl.ds(base, BPW)], pidx_v)
    pltpu.sync_copy(nidx_hbm.at[pl.ds(base, BPW)], nidx_v)
    cp = pltpu.async_copy(feat_hbm.at[pidx_v], pos_v, sem_p)
    cn = pltpu.async_copy(feat_hbm.at[nidx_v], neg_v, sem_n)
    pltpu.sync_copy(feat_hbm.at[pl.ds(base, BPW)], self_v)
    cp.wait()
    cn.wait()

    def row(i, carry):
        ap = jnp.zeros((LANES,), jnp.float32)
        an = jnp.zeros((LANES,), jnp.float32)
        for ch in range(D // LANES):
            s = self_v[i, pl.ds(ch * LANES, LANES)]
            p = pos_v[i, pl.ds(ch * LANES, LANES)]
            m = neg_v[i, pl.ds(ch * LANES, LANES)]
            dp = s - p + EPS
            dn = s - m + EPS
            ap = ap + dp * dp
            an = an + dn * dn
        accp_v[i, :] = ap
        accn_v[i, :] = an
        return carry

    lax.fori_loop(0, BPW, row, 0)
    pltpu.sync_copy(accp_v, outp_hbm.at[pl.ds(base, BPW)])
    pltpu.sync_copy(accn_v, outn_hbm.at[pl.ds(base, BPW)])


@functools.lru_cache(maxsize=1)
def _sc_gather():
  return pl.kernel(
    _sc_body,
    out_type=[
        jax.ShapeDtypeStruct((N, LANES), jnp.float32),
        jax.ShapeDtypeStruct((N, LANES), jnp.float32),
    ],
    mesh=plsc.VectorSubcoreMesh(core_axis_name="c", subcore_axis_name="s"),
    scratch_types=[
        pltpu.VMEM((BPW,), jnp.int32),
        pltpu.VMEM((BPW,), jnp.int32),
        pltpu.VMEM((BPW, D), jnp.float32),
        pltpu.VMEM((BPW, D), jnp.float32),
        pltpu.VMEM((BPW, D), jnp.float32),
        pltpu.VMEM((BPW, LANES), jnp.float32),
        pltpu.VMEM((BPW, LANES), jnp.float32),
        pltpu.SemaphoreType.DMA,
        pltpu.SemaphoreType.DMA,
    ],
  )


def _finish_body(d2p_ref, d2n_ref, out_ref):
    dp = jnp.sqrt(jnp.sum(d2p_ref[:, :], axis=1, keepdims=True))
    dn = jnp.sqrt(jnp.sum(d2n_ref[:, :], axis=1, keepdims=True))
    h = jnp.maximum(MARGIN + dp - dn, 0.0)
    out_ref[0, 0] = jnp.sum(h) / N


_finish = pl.pallas_call(
    _finish_body,
    out_specs=pl.BlockSpec(memory_space=pltpu.SMEM),
    out_shape=jax.ShapeDtypeStruct((1, 1), jnp.float32),
)


def kernel(feature, identity):
    ident = identity.astype(jnp.int32)
    pos2, neg2 = _mine(feature, feature, ident.reshape(N, 1),
                       ident.reshape(1, N))
    d2p, d2n = _sc_gather()(feature, pos2.reshape(N), neg2.reshape(N))
    return _finish(d2p, d2n)[0, 0]
